# baseline (speedup 1.0000x reference)
import functools
import jax
import jax.numpy as jnp
from jax import lax
from jax.experimental import pallas as pl
from jax.experimental.pallas import tpu as pltpu
from jax.experimental.pallas import tpu_sc as plsc

_POOL_SIZE = 200
_HALF = 128
_B = 16384
_K = 64
_NC, _NS = 2, 16
_NW = _NC * _NS
_ROWS_PER_W = _B // _NW
_CHUNK = 128
_NCHUNK = _ROWS_PER_W // _CHUNK
_NBUF = 2
_L = 16


@functools.cache
def _build_neg_sample():
    @functools.partial(
        pl.kernel,
        out_type=jax.ShapeDtypeStruct((_K, _B), jnp.int32),
        mesh=plsc.VectorSubcoreMesh(core_axis_name="c", subcore_axis_name="s",
                                    num_cores=_NC, num_subcores=_NS),
        scratch_types=[
            pltpu.VMEM((_ROWS_PER_W,), jnp.int32),
            pltpu.VMEM((_NBUF, _CHUNK, _HALF), jnp.int32),
            pltpu.VMEM((_NBUF, _CHUNK, _HALF), jnp.int32),
            pltpu.VMEM((_NBUF, _K, _CHUNK), jnp.int32),
            pltpu.VMEM((_NBUF, _K, _CHUNK), jnp.int32),
            pltpu.SemaphoreType.DMA,
            pltpu.SemaphoreType.DMA,
        ],
        compiler_params=pltpu.CompilerParams(use_tc_tiling_on_sc=True,
                                             needs_layout_passes=False,
                                             disable_bounds_checks=True),
    )
    def _neg_sample(uid_hbm, pool_hbm, idxt_hbm, outt_hbm,
                    uid_v, lo_v, hi_v, idx_v, out_v, sem_in, sem_out):
        wid = lax.axis_index("s") * _NC + lax.axis_index("c")
        base_w = wid * _ROWS_PER_W
        pltpu.sync_copy(uid_hbm.at[pl.ds(base_w, _ROWS_PER_W)], uid_v)
        hi_start = pl.multiple_of(wid * 0 + _HALF, _HALF)

        def fire(c):
            buf = c % _NBUF
            base = base_w + c * _CHUNK
            uids = uid_v.at[pl.ds(c * _CHUNK, _CHUNK)]
            pltpu.async_copy(
                pool_hbm.at[uids, pl.ds(0, _HALF)], lo_v.at[buf], sem_in)
            pltpu.async_copy(
                pool_hbm.at[uids, pl.ds(hi_start, _HALF)], hi_v.at[buf],
                sem_in)
            pltpu.async_copy(
                idxt_hbm.at[:, pl.ds(base, _CHUNK)], idx_v.at[buf], sem_in)

        def drain_in(c):
            buf = c % _NBUF
            base = base_w + c * _CHUNK
            uids = uid_v.at[pl.ds(c * _CHUNK, _CHUNK)]
            pltpu.make_async_copy(
                pool_hbm.at[uids, pl.ds(0, _HALF)], lo_v.at[buf],
                sem_in).wait()
            pltpu.make_async_copy(
                pool_hbm.at[uids, pl.ds(hi_start, _HALF)], hi_v.at[buf],
                sem_in).wait()
            pltpu.make_async_copy(
                idxt_hbm.at[:, pl.ds(base, _CHUNK)], idx_v.at[buf],
                sem_in).wait()

        fire(0)
        for c in range(_NCHUNK):
            if c + 1 < _NCHUNK:
                fire(c + 1)
            drain_in(c)
            buf = c % _NBUF
            if c >= _NBUF:
                pltpu.make_async_copy(
                    out_v.at[buf],
                    outt_hbm.at[:, pl.ds(base_w + (c - _NBUF) * _CHUNK,
                                         _CHUNK)], sem_out).wait()

            @plsc.parallel_loop(0, _CHUNK, unroll=16)
            def row_body(b):
                row_sel = jnp.full((_L,), b, jnp.int32)
                for v in range(_K // _L):
                    jrows = lax.iota(jnp.int32, _L) + v * _L
                    cols = plsc.load_gather(idx_v.at[buf], [jrows, row_sel])
                    in_lo = cols < _HALF
                    cols_mod = jnp.bitwise_and(cols, _HALF - 1)
                    vals_lo = plsc.load_gather(lo_v.at[buf],
                                               [row_sel, cols_mod])
                    vals_hi = plsc.load_gather(hi_v.at[buf],
                                               [row_sel, cols_mod])
                    plsc.store_scatter(out_v.at[buf], [jrows, row_sel],
                                       jnp.where(in_lo, vals_lo, vals_hi))
            pltpu.async_copy(
                out_v.at[buf],
                outt_hbm.at[:, pl.ds(base_w + c * _CHUNK, _CHUNK)], sem_out)
        for c in range(max(_NCHUNK - _NBUF, 0), _NCHUNK):
            pltpu.make_async_copy(
                out_v.at[c % _NBUF],
                outt_hbm.at[:, pl.ds(base_w + c * _CHUNK, _CHUNK)],
                sem_out).wait()

    return _neg_sample


def kernel(user_id, pool, idx_k, ones_base):
    neg_t = _build_neg_sample()(user_id, pool, idx_k.T)
    log_q = -jnp.log(_POOL_SIZE * ones_base)
    return neg_t.T, log_q


# speedup vs baseline: 1.2067x; 1.2067x over previous
import functools
import jax
import jax.numpy as jnp
from jax import lax
from jax.experimental import pallas as pl
from jax.experimental.pallas import tpu as pltpu
from jax.experimental.pallas import tpu_sc as plsc

_POOL_SIZE = 200
_HALF = 128
_B = 16384
_K = 64
_NC, _NS = 2, 16
_NW = _NC * _NS
_ROWS_PER_W = _B // _NW
_CHUNK = 128
_NCHUNK = _ROWS_PER_W // _CHUNK
_NBUF = 2
_L = 16


@functools.cache
def _build_neg_sample():
    @functools.partial(
        pl.kernel,
        out_type=jax.ShapeDtypeStruct((_K, _B), jnp.int32),
        mesh=plsc.VectorSubcoreMesh(core_axis_name="c", subcore_axis_name="s",
                                    num_cores=_NC, num_subcores=_NS),
        scratch_types=[
            pltpu.VMEM((_ROWS_PER_W,), jnp.int32),
            pltpu.VMEM((_NBUF, _CHUNK, _HALF), jnp.int32),
            pltpu.VMEM((_NBUF, _CHUNK, _HALF), jnp.int32),
            pltpu.VMEM((_NBUF, _K, _CHUNK), jnp.int32),
            pltpu.VMEM((_NBUF, _K, _CHUNK), jnp.int32),
            pltpu.SemaphoreType.DMA,
            pltpu.SemaphoreType.DMA,
        ],
        compiler_params=pltpu.CompilerParams(use_tc_tiling_on_sc=True,
                                             needs_layout_passes=False,
                                             disable_bounds_checks=True),
    )
    def _neg_sample(uid_hbm, pool_hbm, idxt_hbm, outt_hbm,
                    uid_v, lo_v, hi_v, idx_v, out_v, sem_in, sem_out):
        wid = lax.axis_index("s") * _NC + lax.axis_index("c")
        base_w = wid * _ROWS_PER_W
        pltpu.sync_copy(uid_hbm.at[pl.ds(base_w, _ROWS_PER_W)], uid_v)
        hi_start = pl.multiple_of(wid * 0 + _HALF, _HALF)

        def fire(c):
            buf = c % _NBUF
            base = base_w + c * _CHUNK
            uids = uid_v.at[pl.ds(c * _CHUNK, _CHUNK)]
            pltpu.async_copy(
                pool_hbm.at[uids, pl.ds(0, _HALF)], lo_v.at[buf], sem_in)
            pltpu.async_copy(
                pool_hbm.at[uids, pl.ds(hi_start, _HALF)], hi_v.at[buf],
                sem_in)
            pltpu.async_copy(
                idxt_hbm.at[:, pl.ds(base, _CHUNK)], idx_v.at[buf], sem_in)

        def drain_in(c):
            buf = c % _NBUF
            base = base_w + c * _CHUNK
            uids = uid_v.at[pl.ds(c * _CHUNK, _CHUNK)]
            pltpu.make_async_copy(
                pool_hbm.at[uids, pl.ds(0, _HALF)], lo_v.at[buf],
                sem_in).wait()
            pltpu.make_async_copy(
                pool_hbm.at[uids, pl.ds(hi_start, _HALF)], hi_v.at[buf],
                sem_in).wait()
            pltpu.make_async_copy(
                idxt_hbm.at[:, pl.ds(base, _CHUNK)], idx_v.at[buf],
                sem_in).wait()

        fire(0)
        for c in range(_NCHUNK):
            if c + 1 < _NCHUNK:
                fire(c + 1)
            drain_in(c)
            buf = c % _NBUF
            if c >= _NBUF:
                pltpu.make_async_copy(
                    out_v.at[buf],
                    outt_hbm.at[:, pl.ds(base_w + (c - _NBUF) * _CHUNK,
                                         _CHUNK)], sem_out).wait()

            @plsc.parallel_loop(0, _K * (_CHUNK // _L), unroll=8)
            def row_body(t):
                j = lax.shift_right_logical(t, 3)
                bb = jnp.bitwise_and(t, (_CHUNK // _L) - 1) * _L
                brows = lax.iota(jnp.int32, _L) + bb
                cols = idx_v[buf, j, pl.ds(bb, _L)]
                in_lo = cols < _HALF
                cols_mod = jnp.bitwise_and(cols, _HALF - 1)
                vals_lo = plsc.load_gather(lo_v.at[buf], [brows, cols_mod])
                vals_hi = plsc.load_gather(hi_v.at[buf], [brows, cols_mod])
                out_v[buf, j, pl.ds(bb, _L)] = jnp.where(
                    in_lo, vals_lo, vals_hi)
            pltpu.async_copy(
                out_v.at[buf],
                outt_hbm.at[:, pl.ds(base_w + c * _CHUNK, _CHUNK)], sem_out)
        for c in range(max(_NCHUNK - _NBUF, 0), _NCHUNK):
            pltpu.make_async_copy(
                out_v.at[c % _NBUF],
                outt_hbm.at[:, pl.ds(base_w + c * _CHUNK, _CHUNK)],
                sem_out).wait()

    return _neg_sample


def kernel(user_id, pool, idx_k, ones_base):
    neg_t = _build_neg_sample()(user_id, pool, idx_k.T)
    log_q = -jnp.log(_POOL_SIZE * ones_base)
    return neg_t.T, log_q
